# baseline (speedup 1.0000x reference)
import jax
import jax.numpy as jnp
from jax import lax
from jax.experimental import pallas as pl
from jax.experimental.pallas import tpu as pltpu
from jax.experimental.pallas import tpu_sc as plsc

_ENT_BLOCK = 25000
_N_REL = 1000
_D = 128
_REL_ROWS = 128


def _tc_ent_body(ent_in, ent_out):
    ent_out[...] = ent_in[...]


def _sc_rel_body(rel_in, rel_out, relbuf, rsem):
    wid = lax.axis_index("s") * 2 + lax.axis_index("c")

    @pl.when(wid < 8)
    def _():
        rbase = jnp.minimum(wid * _REL_ROWS, _N_REL - _REL_ROWS)
        cin = pltpu.make_async_copy(
            rel_in.at[pl.ds(rbase, _REL_ROWS)], relbuf, rsem)
        cin.start()
        cin.wait()
        cout = pltpu.make_async_copy(
            relbuf, rel_out.at[pl.ds(rbase, _REL_ROWS)], rsem)
        cout.start()
        cout.wait()


def kernel(entity_weight, relation_weight):
    n_ent, d = entity_weight.shape
    grid = n_ent // _ENT_BLOCK
    rel_run = pl.kernel(
        _sc_rel_body,
        out_type=jax.ShapeDtypeStruct((_N_REL, _D), jnp.float32),
        mesh=plsc.VectorSubcoreMesh(core_axis_name="c", subcore_axis_name="s"),
        scratch_types=[
            pltpu.VMEM((_REL_ROWS, _D), jnp.float32),
            pltpu.SemaphoreType.DMA,
        ],
    )
    rel_out = rel_run(relation_weight)
    ent_out = pl.pallas_call(
        _tc_ent_body,
        grid=(grid,),
        in_specs=[pl.BlockSpec((_ENT_BLOCK, d), lambda i: (i, 0))],
        out_specs=pl.BlockSpec((_ENT_BLOCK, d), lambda i: (i, 0)),
        out_shape=jax.ShapeDtypeStruct(entity_weight.shape, entity_weight.dtype),
    )(entity_weight)
    return (ent_out, rel_out)


# speedup vs baseline: 1.4738x; 1.4738x over previous
import jax
import jax.numpy as jnp
from jax.experimental import pallas as pl
from jax.experimental.pallas import tpu as pltpu

_ENT_BLOCK = 25000


def _copy_body(ent_in, rel_in, ent_out, rel_out):
    ent_out[...] = ent_in[...]

    @pl.when(pl.program_id(0) == 0)
    def _():
        rel_out[...] = rel_in[...]


def kernel(entity_weight, relation_weight):
    n_ent, d = entity_weight.shape
    n_rel, _ = relation_weight.shape
    grid = n_ent // _ENT_BLOCK
    ent_out, rel_out = pl.pallas_call(
        _copy_body,
        grid=(grid,),
        in_specs=[
            pl.BlockSpec((_ENT_BLOCK, d), lambda i: (i, 0)),
            pl.BlockSpec((n_rel, d), lambda i: (0, 0)),
        ],
        out_specs=[
            pl.BlockSpec((_ENT_BLOCK, d), lambda i: (i, 0)),
            pl.BlockSpec((n_rel, d), lambda i: (0, 0)),
        ],
        out_shape=[
            jax.ShapeDtypeStruct(entity_weight.shape, entity_weight.dtype),
            jax.ShapeDtypeStruct(relation_weight.shape, relation_weight.dtype),
        ],
    )(entity_weight, relation_weight)
    return (ent_out, rel_out)
